# baseline (speedup 1.0000x reference)
import jax
import jax.numpy as jnp
from jax.experimental import pallas as pl
from jax.experimental.pallas import tpu as pltpu

NUM_CLASSES = 10000
CLASSES_PER_TASK = 1000
CURRENT_TASK = 1
BAND_START = CURRENT_TASK * CLASSES_PER_TASK
BAND_END = BAND_START + CLASSES_PER_TASK

ROWS_PER_BLOCK = 256


def _band_affine_kernel(a_ref, b_ref, x_ref, o_ref):
    o_ref[...] = x_ref[...] * a_ref[...] + b_ref[...]


def kernel(x, alpha, beta):
    m, n = x.shape
    col = jnp.arange(n, dtype=jnp.int32)
    in_band = (col >= BAND_START) & (col < BAND_END)
    a_vec = jnp.where(in_band, alpha[0], jnp.float32(1.0))[None, :]
    b_vec = jnp.where(in_band, beta[0], jnp.float32(0.0))[None, :]
    grid = (m // ROWS_PER_BLOCK,)
    return pl.pallas_call(
        _band_affine_kernel,
        grid=grid,
        in_specs=[
            pl.BlockSpec((1, n), lambda i: (0, 0)),
            pl.BlockSpec((1, n), lambda i: (0, 0)),
            pl.BlockSpec((ROWS_PER_BLOCK, n), lambda i: (i, 0)),
        ],
        out_specs=pl.BlockSpec((ROWS_PER_BLOCK, n), lambda i: (i, 0)),
        out_shape=jax.ShapeDtypeStruct((m, n), x.dtype),
    )(a_vec, b_vec, x)


# speedup vs baseline: 1.0148x; 1.0148x over previous
import jax
import jax.numpy as jnp
from jax.experimental import pallas as pl
from jax.experimental.pallas import tpu as pltpu

NUM_CLASSES = 10000
CLASSES_PER_TASK = 1000
CURRENT_TASK = 1
BAND_START = CURRENT_TASK * CLASSES_PER_TASK
BAND_END = BAND_START + CLASSES_PER_TASK

ROWS_PER_BLOCK = 256


def _band_affine_kernel(a_ref, b_ref, x_ref, o_ref):
    o_ref[...] = x_ref[...] * a_ref[...] + b_ref[...]


def kernel(x, alpha, beta):
    m, n = x.shape
    col = jnp.arange(n, dtype=jnp.int32)
    in_band = (col >= BAND_START) & (col < BAND_END)
    a_vec = jnp.where(in_band, alpha[0], jnp.float32(1.0))[None, :]
    b_vec = jnp.where(in_band, beta[0], jnp.float32(0.0))[None, :]
    grid = (m // ROWS_PER_BLOCK,)
    return pl.pallas_call(
        _band_affine_kernel,
        grid=grid,
        in_specs=[
            pl.BlockSpec((1, n), lambda i: (0, 0)),
            pl.BlockSpec((1, n), lambda i: (0, 0)),
            pl.BlockSpec((ROWS_PER_BLOCK, n), lambda i: (i, 0)),
        ],
        out_specs=pl.BlockSpec((ROWS_PER_BLOCK, n), lambda i: (i, 0)),
        out_shape=jax.ShapeDtypeStruct((m, n), x.dtype),
        compiler_params=pltpu.CompilerParams(
            dimension_semantics=("parallel",),
        ),
    )(a_vec, b_vec, x)
